# Initial kernel scaffold; baseline (speedup 1.0000x reference)
import jax
import jax.numpy as jnp
from jax.experimental import pallas as pl


def kernel(x, vec, edge_index, r_ij, f_ij, d_ij, Wq, bq, Wk, bk, Wv, bv, Wo, bo, Wvec, Wdk, bdk, Wdv, bdv, ln_g, ln_b):
    raise NotImplementedError("write your pallas kernel here")


# speedup vs baseline: 1.0000x; 1.0000x over previous
import jax
import jax.numpy as jnp
from jax.experimental import pallas as pl

N = 10000
E = 320000
L = 2
C = 128
XH = 128
VC = 64
EA = 16
H = 8
CUTOFF = 5.0


def _cosine_cutoff(r):
    return 0.5 * (jnp.cos(r * jnp.pi / CUTOFF) + 1.0) * (r < CUTOFF).astype(r.dtype)


def _layernorm(x, g, b):
    m = jnp.mean(x, axis=-1, keepdims=True)
    v = jnp.var(x, axis=-1, keepdims=True)
    return (x - m) / jnp.sqrt(v + 1e-5) * g + b


def kernel(x, vec, edge_index, r_ij, f_ij, d_ij, Wq, bq, Wk, bk, Wv, bv, Wo, bo,
           Wvec, Wdk, bdk, Wdv, bdv, ln_g, ln_b):
    hd = XH // H
    vd = VC // H
    src = edge_index[0]
    dst = edge_index[1]
    n = x.shape[0]
    cut = _cosine_cutoff(r_ij)
    for l in range(L):
        h = _layernorm(x, ln_g[l], ln_b[l])
        q = (h @ Wq[l] + bq[l]).reshape(n, H, hd)
        k = (h @ Wk[l] + bk[l]).reshape(n, H, hd)
        v = (h @ Wv[l] + bv[l]).reshape(n, H, hd + 2 * vd)
        vp = jnp.einsum('ndc,co->ndo', vec, Wvec[l])
        vec1 = vp[..., :C]
        vec2 = vp[..., C:2 * C]
        vec3 = vp[..., 2 * C:]
        vec_r = vec.reshape(n, 3, H, vd)
        vec_dot = jnp.sum(vec1 * vec2, axis=1)
        dk = jax.nn.silu(f_ij @ Wdk[l] + bdk[l]).reshape(-1, H, hd)
        dv = jax.nn.silu(f_ij @ Wdv[l] + bdv[l]).reshape(-1, H, hd + 2 * vd)
        attn = jnp.sum(jnp.take(q, dst, axis=0) * jnp.take(k, src, axis=0) * dk, axis=-1)
        attn = jax.nn.silu(attn) * cut[:, None]
        vj = jnp.take(v, src, axis=0) * dv
        xm = vj[:, :, :hd] * attn[:, :, None]
        v1 = vj[:, :, hd:hd + vd]
        v2 = vj[:, :, hd + vd:]
        vecm = jnp.take(vec_r, src, axis=0) * v1[:, None, :, :] + v2[:, None, :, :] * d_ij[:, :, None, None]
        xagg = jax.ops.segment_sum(xm, dst, num_segments=n).reshape(n, XH)
        vagg = jax.ops.segment_sum(vecm, dst, num_segments=n).reshape(n, 3, VC)
        o = xagg @ Wo[l] + bo[l]
        o1 = o[:, :VC]
        o2 = o[:, VC:VC + C]
        o3 = o[:, VC + C:]
        dx = vec_dot * o2 + o3
        dvec = vec3 * o1[:, None, :] + vagg
        x = x + dx
        vec = vec + dvec
    return (x, vec)


# speedup vs baseline: 13.6986x; 13.6981x over previous
import functools

import jax
import jax.numpy as jnp
import numpy as np
from jax import lax
from jax.experimental import pallas as pl
from jax.experimental.pallas import tpu as pltpu
from jax.experimental.pallas import tpu_sc as plsc

N = 10000
E = 320000
L = 2
C = 128
XH = 128
VC = 64
EA = 16
H = 8
HD = XH // H
VD = VC // H
CUTOFF = 5.0

NT = 16
EPT = E // NT
B = 80
NCH = EPT // B
ZR = 40
NZC = N // ZR
ZRND = (NZC + NT - 1) // NT

F32 = jnp.float32


_RB = 400
_EB = 2000


def _node_pre_body(x_ref, v3_ref, g_ref, b_ref, wq_ref, bq_ref, wk_ref, bk_ref,
                   wvx_ref, bvx_ref, wv12_ref, bv12_ref, w1_ref, w2_ref, w3_ref,
                   q_ref, k_ref, vx_ref, v12_ref, vdot_ref, vec3_ref):
    xb = x_ref[...]
    m = jnp.mean(xb, axis=-1, keepdims=True)
    var = jnp.mean((xb - m) ** 2, axis=-1, keepdims=True)
    h = (xb - m) * lax.rsqrt(var + 1e-5) * g_ref[...] + b_ref[...]
    dot = lambda a, w: lax.dot(a, w, preferred_element_type=F32)
    q_ref[...] = dot(h, wq_ref[...]) + bq_ref[...]
    k_ref[...] = dot(h, wk_ref[...]) + bk_ref[...]
    vx_ref[...] = dot(h, wvx_ref[...]) + bvx_ref[...]
    v12_ref[...] = dot(h, wv12_ref[...]) + bv12_ref[...]
    vb = v3_ref[...]
    vec1 = dot(vb, w1_ref[...])
    vec2 = dot(vb, w2_ref[...])
    vec3_ref[...] = dot(vb, w3_ref[...])
    p = (vec1 * vec2).reshape(_RB, 3, C)
    vdot_ref[...] = jnp.sum(p, axis=1)


def _node_pre(x, vec3d, lng, lnb, wq, bq, wk, bk, wvx, bvx, wv12, bv12, w1, w2, w3):
    g = N // _RB
    row = pl.BlockSpec((_RB, C), lambda i: (i, 0))
    row3 = pl.BlockSpec((3 * _RB, VC), lambda i: (i, 0))
    full = lambda s: pl.BlockSpec(s, lambda i: (0, 0))
    return pl.pallas_call(
        _node_pre_body,
        grid=(g,),
        in_specs=[row, row3, full((1, C)), full((1, C)),
                  full((C, XH)), full((1, XH)), full((C, XH)), full((1, XH)),
                  full((C, XH)), full((1, XH)), full((C, XH)), full((1, XH)),
                  full((VC, C)), full((VC, C)), full((VC, VC))],
        out_specs=[row, row, row, row, row, row3],
        out_shape=[jax.ShapeDtypeStruct((N, XH), F32)] * 5
        + [jax.ShapeDtypeStruct((3 * N, VC), F32)],
    )(x, vec3d, lng, lnb, wq, bq, wk, bk, wvx, bvx, wv12, bv12, w1, w2, w3)


def _edge_pre_body(f_ref, r_ref, wdk_ref, bdk_ref, wdvx_ref, bdvx_ref,
                   wdv12_ref, bdv12_ref, dk_ref, dvx_ref, dv12_ref):
    f = f_ref[...]
    r = r_ref[...]
    cut = 0.5 * (jnp.cos(r * (jnp.pi / CUTOFF)) + 1.0) * (r < CUTOFF).astype(F32)
    dot = lambda a, w: lax.dot(a, w, preferred_element_type=F32)
    silu = lambda t: t * (1.0 / (1.0 + jnp.exp(-t)))
    dk_ref[...] = silu(dot(f, wdk_ref[...]) + bdk_ref[...])
    dvx_ref[...] = silu(dot(f, wdvx_ref[...]) + bdvx_ref[...]) * cut
    dv12_ref[...] = silu(dot(f, wdv12_ref[...]) + bdv12_ref[...])


def _edge_pre(f_ij, r2d, wdk, bdk, wdvx, bdvx, wdv12, bdv12):
    g = E // _EB
    row = pl.BlockSpec((_EB, XH), lambda i: (i, 0))
    full = lambda s: pl.BlockSpec(s, lambda i: (0, 0))
    return pl.pallas_call(
        _edge_pre_body,
        grid=(g,),
        in_specs=[pl.BlockSpec((_EB, EA), lambda i: (i, 0)),
                  pl.BlockSpec((_EB, 1), lambda i: (i, 0)),
                  full((EA, XH)), full((1, XH)), full((EA, XH)), full((1, XH)),
                  full((EA, XH)), full((1, XH))],
        out_specs=[row, row, row],
        out_shape=[jax.ShapeDtypeStruct((E, XH), F32)] * 3,
    )(f_ij, r2d, wdk, bdk, wdvx, bdvx, wdv12, bdv12)


def _node_post_body(xagg_ref, xaggb_ref, vdot_ref, vec3_ref, vagg_ref,
                    vaggb_ref, x_ref, vec_ref,
                    wo1_ref, bo1_ref, wo2_ref, bo2_ref, wo3_ref, bo3_ref,
                    xn_ref, vn_ref):
    xagg = xagg_ref[...] + xaggb_ref[...]
    dot = lambda a, w: lax.dot(a, w, preferred_element_type=F32)
    o1 = dot(xagg, wo1_ref[...]) + bo1_ref[...]
    o2 = dot(xagg, wo2_ref[...]) + bo2_ref[...]
    o3 = dot(xagg, wo3_ref[...]) + bo3_ref[...]
    xn_ref[...] = x_ref[...] + vdot_ref[...] * o2 + o3
    o1r = jnp.broadcast_to(o1[:, None, :], (_RB, 3, VC)).reshape(3 * _RB, VC)
    vn_ref[...] = vec_ref[...] + vec3_ref[...] * o1r + vagg_ref[...] + vaggb_ref[...]


def _node_post(xagg2, vdot, vec3, vagg2, x, vec3d, wo1, bo1, wo2, bo2, wo3, bo3):
    g = N // _RB
    row = pl.BlockSpec((_RB, C), lambda i: (i, 0))
    rowb = pl.BlockSpec((_RB, C), lambda i: (i + g, 0))
    row3 = pl.BlockSpec((3 * _RB, VC), lambda i: (i, 0))
    row3b = pl.BlockSpec((3 * _RB, VC), lambda i: (i + g, 0))
    full = lambda s: pl.BlockSpec(s, lambda i: (0, 0))
    return pl.pallas_call(
        _node_post_body,
        grid=(g,),
        in_specs=[row, rowb, row, row3, row3, row3b, row, row3,
                  full((XH, VC)), full((1, VC)), full((XH, C)), full((1, C)),
                  full((XH, C)), full((1, C))],
        out_specs=[row, row3],
        out_shape=[jax.ShapeDtypeStruct((N, C), F32),
                   jax.ShapeDtypeStruct((3 * N, VC), F32)],
    )(xagg2, xagg2, vdot, vec3, vagg2, vagg2, x, vec3d,
      wo1, bo1, wo2, bo2, wo3, bo3)


NW = 2 * NT
EPW = E // NW

BA = 40
NCA = EPW // BA
BV = 16
NCV = EPW // BV

ZRA = 40
ZRV = 16


def _sc_params():
    return pltpu.CompilerParams(needs_layout_passes=False,
                                use_tc_tiling_on_sc=False)


def _mesh():
    return plsc.VectorSubcoreMesh(core_axis_name="c", subcore_axis_name="s",
                                  num_cores=2, num_subcores=NT)


def _acc_sweep(s, acc, buf, rows, out_ref=None, out_base=0):
    nch = N // rows
    rnd = (nch + NT - 1) // NT

    def _step(z, _):
        t = z * NT + s

        @pl.when(t < nch)
        def _():
            r0 = t * rows
            if out_ref is None:
                pltpu.sync_copy(buf, acc.at[pl.ds(r0, rows)])
            else:
                pltpu.sync_copy(acc.at[pl.ds(r0, rows)], buf)
                pltpu.sync_copy(buf, out_ref.at[pl.ds(out_base + r0, rows)])
        return 0

    lax.fori_loop(0, rnd, _step, 0)


@functools.lru_cache(maxsize=1)
def _build_sc_xagg():
    return pl.kernel(
        _sc_xagg_body,
        out_type=jax.ShapeDtypeStruct((2 * N, 128), F32),
        mesh=_mesh(),
        scratch_types=[
            pltpu.VMEM((BA,), jnp.int32),
            pltpu.VMEM((BA,), jnp.int32),
            pltpu.VMEM((BA, 128), F32),
            pltpu.VMEM((BA, 128), F32),
            pltpu.VMEM((BA, 128), F32),
            pltpu.VMEM((BA, 128), F32),
            pltpu.VMEM((BA, 128), F32),
            pltpu.VMEM((BA, 128), F32),
            pltpu.VMEM((16,), F32),
            pltpu.VMEM_SHARED((N, 128), F32),
            pltpu.SemaphoreType.DMA,
        ],
        compiler_params=_sc_params(),
    )


def _sc_xagg_body(q_hbm, k_hbm, vx_hbm, dk_hbm, dvx_hbm, src_hbm, dst_hbm,
                  out_hbm, idx_src, idx_dst, rq, rk, rv, rdk, rdv,
                  msg, shuf, acc, sem):
    c = lax.axis_index("c")
    s = lax.axis_index("s")
    zero16 = jnp.zeros((16,), F32)
    iota = lax.broadcasted_iota(jnp.int32, (16,), 0)

    def _zb(i, _):
        for j in range(8):
            msg[i, pl.ds(16 * j, 16)] = zero16
        return 0

    lax.fori_loop(0, BA, _zb, 0)
    _acc_sweep(s, acc, msg, ZRA)
    plsc.subcore_barrier()

    base0 = (c * NT + s) * EPW

    def _chunk(i, _):
        base = base0 + i * BA
        pltpu.sync_copy(src_hbm.at[pl.ds(base, BA)], idx_src)
        pltpu.sync_copy(dst_hbm.at[pl.ds(base, BA)], idx_dst)
        gq = pltpu.async_copy(q_hbm.at[idx_dst], rq, sem)
        gk = pltpu.async_copy(k_hbm.at[idx_src], rk, sem)
        gv = pltpu.async_copy(vx_hbm.at[idx_src], rv, sem)
        pltpu.sync_copy(dk_hbm.at[pl.ds(base, BA)], rdk)
        pltpu.sync_copy(dvx_hbm.at[pl.ds(base, BA)], rdv)
        gq.wait()
        gk.wait()
        gv.wait()

        def _edge(e, _):
            for h in range(H):
                sl = pl.ds(HD * h, HD)
                av = rq[e, sl] * rk[e, sl] * rdk[e, sl]
                for sh in (8, 4, 2, 1):
                    idx = jnp.bitwise_and(iota + sh, 15)
                    plsc.store_scatter(shuf, [idx], av)
                    av = av + shuf[...]
                sig = 1.0 / (1.0 + jnp.exp(-av))
                msg[e, sl] = rv[e, sl] * rdv[e, sl] * (av * sig)
            return 0

        lax.fori_loop(0, BA, _edge, 0)
        pltpu.sync_copy(msg, acc.at[idx_dst], add=True)
        return 0

    lax.fori_loop(0, NCA, _chunk, 0)
    plsc.subcore_barrier()
    _acc_sweep(s, acc, msg, ZRA, out_hbm, c * N)


@functools.lru_cache(maxsize=1)
def _build_sc_vagg():
    return pl.kernel(
        _sc_vagg_body,
        out_type=jax.ShapeDtypeStruct((2 * N, 192), F32),
        mesh=_mesh(),
        scratch_types=[
            pltpu.VMEM((BV,), jnp.int32),
            pltpu.VMEM((BV,), jnp.int32),
            pltpu.VMEM((BV, 192), F32),
            pltpu.VMEM((BV, 128), F32),
            pltpu.VMEM((BV, 128), F32),
            pltpu.VMEM((BV, 16), F32),
            pltpu.VMEM((BV, 192), F32),
            pltpu.VMEM_SHARED((N, 192), F32),
            pltpu.SemaphoreType.DMA,
        ],
        compiler_params=_sc_params(),
    )


def _sc_vagg_body(vecf_hbm, v12_hbm, dv12_hbm, d16_hbm, src_hbm, dst_hbm,
                  out_hbm, idx_src, idx_dst, rvec, rv12, rdv, rd, msg,
                  acc, sem):
    c = lax.axis_index("c")
    s = lax.axis_index("s")
    zero16 = jnp.zeros((16,), F32)

    def _zb(i, _):
        for j in range(12):
            msg[i, pl.ds(16 * j, 16)] = zero16
        return 0

    lax.fori_loop(0, BV, _zb, 0)
    _acc_sweep(s, acc, msg, ZRV)
    plsc.subcore_barrier()

    base0 = (c * NT + s) * EPW

    def _chunk(i, _):
        base = base0 + i * BV
        pltpu.sync_copy(src_hbm.at[pl.ds(base, BV)], idx_src)
        pltpu.sync_copy(dst_hbm.at[pl.ds(base, BV)], idx_dst)
        gvec = pltpu.async_copy(vecf_hbm.at[idx_src], rvec, sem)
        gv12 = pltpu.async_copy(v12_hbm.at[idx_src], rv12, sem)
        pltpu.sync_copy(dv12_hbm.at[pl.ds(base, BV)], rdv)
        pltpu.sync_copy(d16_hbm.at[pl.ds(base, BV)], rd)
        gvec.wait()
        gv12.wait()

        def _edge(e, _):
            drow = rd[e, :]
            for j in range(4):
                slj = pl.ds(16 * j, 16)
                sl2 = pl.ds(64 + 16 * j, 16)
                v1 = rv12[e, slj] * rdv[e, slj]
                v2 = rv12[e, sl2] * rdv[e, sl2]
                for d in range(3):
                    slo = pl.ds(64 * d + 16 * j, 16)
                    msg[e, slo] = rvec[e, slo] * v1 + v2 * drow[d]
            return 0

        lax.fori_loop(0, BV, _edge, 0)
        pltpu.sync_copy(msg, acc.at[idx_dst], add=True)
        return 0

    lax.fori_loop(0, NCV, _chunk, 0)
    plsc.subcore_barrier()
    _acc_sweep(s, acc, msg, ZRV, out_hbm, c * N)


_VX_COLS = (np.arange(H)[:, None] * (HD + 2 * VD) + np.arange(HD)[None, :]).reshape(-1)
_V1_COLS = (np.arange(H)[:, None] * (HD + 2 * VD) + HD + np.arange(VD)[None, :]).reshape(-1)
_V2_COLS = (np.arange(H)[:, None] * (HD + 2 * VD) + HD + VD + np.arange(VD)[None, :]).reshape(-1)
_V12_COLS = np.concatenate([_V1_COLS, _V2_COLS])


def kernel(x, vec, edge_index, r_ij, f_ij, d_ij, Wq, bq, Wk, bk, Wv, bv, Wo, bo,
           Wvec, Wdk, bdk, Wdv, bdv, ln_g, ln_b):
    src = edge_index[0].astype(jnp.int32)
    dst = edge_index[1].astype(jnp.int32)
    r2d = r_ij.reshape(E, 1)
    d4 = jnp.concatenate([d_ij, jnp.zeros((E, 13), F32)], axis=1)

    vec3d = vec.reshape(3 * N, VC)

    for l in range(L):
        vecf = vec3d.reshape(N, 3 * VC)
        wvx = Wv[l][:, _VX_COLS]
        bvx = bv[l][_VX_COLS].reshape(1, XH)
        wv12 = Wv[l][:, _V12_COLS]
        bv12 = bv[l][_V12_COLS].reshape(1, XH)
        wdvx = Wdv[l][:, _VX_COLS]
        bdvx = bdv[l][_VX_COLS].reshape(1, XH)
        wdv12 = Wdv[l][:, _V12_COLS]
        bdv12 = bdv[l][_V12_COLS].reshape(1, XH)

        q, k, vx, v12, vdot, vec3 = _node_pre(
            x, vec3d, ln_g[l].reshape(1, C), ln_b[l].reshape(1, C),
            Wq[l], bq[l].reshape(1, XH), Wk[l], bk[l].reshape(1, XH),
            wvx, bvx, wv12, bv12,
            Wvec[l][:, :C], Wvec[l][:, C:2 * C], Wvec[l][:, 2 * C:])

        dk, dvx, dv12 = _edge_pre(
            f_ij, r2d, Wdk[l], bdk[l].reshape(1, XH), wdvx, bdvx, wdv12, bdv12)

        xagg2 = _build_sc_xagg()(q, k, vx, dk, dvx, src, dst)
        vagg2 = _build_sc_vagg()(vecf, v12, dv12, d4, src, dst).reshape(6 * N, VC)

        x, vec3d = _node_post(
            xagg2, vdot, vec3, vagg2, x, vec3d,
            Wo[l][:, :VC], bo[l][:VC].reshape(1, VC),
            Wo[l][:, VC:VC + C], bo[l][VC:VC + C].reshape(1, C),
            Wo[l][:, VC + C:], bo[l][VC + C:].reshape(1, C))

    return (x, vec3d.reshape(N, 3, VC))
